# Initial kernel scaffold; baseline (speedup 1.0000x reference)
import functools

import jax
import jax.numpy as jnp
from jax import lax
from jax.experimental import pallas as pl
from jax.experimental.pallas import tpu as pltpu
from jax.experimental.pallas import tpu_sc as plsc

_NT = 4
_V = 100000
_D = 64
_BT = 150000
_TOTAL = _NT * _BT

_NW = 32
_SUB = 120
_KS = 8
_CH = _SUB * _KS
_NCHUNKS = _TOTAL // _CH

_mesh = plsc.VectorSubcoreMesh(core_axis_name="c", subcore_axis_name="s")


@functools.partial(
    pl.kernel,
    mesh=_mesh,
    out_type=jax.ShapeDtypeStruct((_TOTAL, _D), jnp.float32),
    scratch_types=[
        pltpu.VMEM((_CH,), jnp.int32),
        pltpu.VMEM((_CH, _D), jnp.float32),
        pltpu.SemaphoreType.DMA,
    ],
)
def _gather_kernel(feats_hbm, idx_hbm, out_hbm, idx_v, rows_v, gsem):
    info = plsc.get_sparse_core_info()
    nc = info.num_cores
    wid = lax.axis_index("s") * nc + lax.axis_index("c")
    count = (_NCHUNKS - wid + _NW - 1) // _NW

    def body(i, carry):
        c = wid + i * _NW
        base = c * _CH
        pltpu.sync_copy(idx_hbm.at[pl.ds(base, _CH)], idx_v)
        handles = []
        for j in range(_KS):
            handles.append(
                pltpu.async_copy(
                    feats_hbm.at[idx_v.at[pl.ds(j * _SUB, _SUB)]],
                    rows_v.at[pl.ds(j * _SUB, _SUB)],
                    gsem,
                )
            )
        for h in handles:
            h.wait()
        pltpu.sync_copy(rows_v, out_hbm.at[pl.ds(base, _CH)])
        return carry

    lax.fori_loop(0, count, body, 0)


def kernel(features_list, index_map_list):
    feats = features_list.reshape(_NT * _V, _D)
    offs = (jnp.arange(_NT, dtype=jnp.int32) * _V)[:, None]
    idx = (index_map_list.astype(jnp.int32) + offs).reshape(_TOTAL)
    return _gather_kernel(feats, idx)


# speedup vs baseline: 1.5046x; 1.5046x over previous
import functools

import jax
import jax.numpy as jnp
from jax import lax
from jax.experimental import pallas as pl
from jax.experimental.pallas import tpu as pltpu
from jax.experimental.pallas import tpu_sc as plsc

_NT = 4
_V = 100000
_D = 64
_BT = 150000
_TOTAL = _NT * _BT

_NW = 32
_SUB = 120
_KS = 8
_CH = _SUB * _KS
_NCHUNKS = _TOTAL // _CH

_mesh = plsc.VectorSubcoreMesh(core_axis_name="c", subcore_axis_name="s")


@functools.partial(
    pl.kernel,
    mesh=_mesh,
    out_type=jax.ShapeDtypeStruct((_TOTAL, _D), jnp.float32),
    scratch_types=[
        pltpu.VMEM((_CH,), jnp.int32),
        pltpu.VMEM((_CH, _D), jnp.float32),
        pltpu.SemaphoreType.DMA,
    ],
    compiler_params=pltpu.CompilerParams(use_tc_tiling_on_sc=False),
)
def _gather_kernel(feats_hbm, idx_hbm, out_hbm, idx_v, rows_v, gsem):
    info = plsc.get_sparse_core_info()
    nc = info.num_cores
    wid = lax.axis_index("s") * nc + lax.axis_index("c")
    count = (_NCHUNKS - wid + _NW - 1) // _NW

    def body(i, carry):
        c = wid + i * _NW
        base = c * _CH
        pltpu.sync_copy(idx_hbm.at[pl.ds(base, _CH)], idx_v)
        handles = []
        for j in range(_KS):
            handles.append(
                pltpu.async_copy(
                    feats_hbm.at[idx_v.at[pl.ds(j * _SUB, _SUB)]],
                    rows_v.at[pl.ds(j * _SUB, _SUB)],
                    gsem,
                )
            )
        for h in handles:
            h.wait()
        pltpu.sync_copy(rows_v, out_hbm.at[pl.ds(base, _CH)])
        return carry

    lax.fori_loop(0, count, body, 0)


def kernel(features_list, index_map_list):
    feats = features_list.reshape(_NT * _V, _D)
    offs = (jnp.arange(_NT, dtype=jnp.int32) * _V)[:, None]
    idx = (index_map_list.astype(jnp.int32) + offs).reshape(_TOTAL)
    return _gather_kernel(feats, idx)


# speedup vs baseline: 1.5290x; 1.0162x over previous
import functools

import jax
import jax.numpy as jnp
from jax import lax
from jax.experimental import pallas as pl
from jax.experimental.pallas import tpu as pltpu
from jax.experimental.pallas import tpu_sc as plsc

_NT = 4
_V = 100000
_D = 64
_BT = 150000
_TOTAL = _NT * _BT

_NW = 32
_SUB = 120
_KS = 8
_CH = _SUB * _KS
_NCHUNKS = _TOTAL // _CH
_KMAX = (_NCHUNKS + _NW - 1) // _NW

_mesh = plsc.VectorSubcoreMesh(core_axis_name="c", subcore_axis_name="s")


@functools.partial(
    pl.kernel,
    mesh=_mesh,
    out_type=jax.ShapeDtypeStruct((_TOTAL, _D), jnp.float32),
    scratch_types=[
        pltpu.VMEM((_CH,), jnp.int32),
        pltpu.VMEM((_CH,), jnp.int32),
        pltpu.VMEM((_CH, _D), jnp.float32),
        pltpu.VMEM((_CH, _D), jnp.float32),
        pltpu.SemaphoreType.DMA,
        pltpu.SemaphoreType.DMA,
        pltpu.SemaphoreType.DMA,
    ],
    compiler_params=pltpu.CompilerParams(use_tc_tiling_on_sc=False),
)
def _gather_kernel(feats_hbm, idx_hbm, out_hbm,
                   idx0, idx1, rows0, rows1, gsem, osem0, osem1):
    info = plsc.get_sparse_core_info()
    nc = info.num_cores
    wid = lax.axis_index("s") * nc + lax.axis_index("c")
    count = (_NCHUNKS - wid + _NW - 1) // _NW

    idx_bufs = (idx0, idx1)
    row_bufs = (rows0, rows1)
    osems = (osem0, osem1)

    def do_chunk(k, t):
        idx_v = idx_bufs[t]
        rows_v = row_bufs[t]
        osem = osems[t]
        c = wid + k * _NW
        base = c * _CH

        @pl.when(k < count)
        def _():
            @pl.when(k >= 2)
            def _():
                pltpu.make_async_copy(
                    rows_v, out_hbm.at[pl.ds(0, _CH)], osem
                ).wait()

            pltpu.sync_copy(idx_hbm.at[pl.ds(base, _CH)], idx_v)
            handles = []
            for j in range(_KS):
                handles.append(
                    pltpu.async_copy(
                        feats_hbm.at[idx_v.at[pl.ds(j * _SUB, _SUB)]],
                        rows_v.at[pl.ds(j * _SUB, _SUB)],
                        gsem,
                    )
                )
            for h in handles:
                h.wait()
            pltpu.async_copy(rows_v, out_hbm.at[pl.ds(base, _CH)], osem)

    def pair_body(p, carry):
        do_chunk(2 * p, 0)
        do_chunk(2 * p + 1, 1)
        return carry

    lax.fori_loop(0, (_KMAX + 1) // 2, pair_body, 0)

    pltpu.make_async_copy(rows0, out_hbm.at[pl.ds(0, _CH)], osem0).wait()
    pltpu.make_async_copy(rows1, out_hbm.at[pl.ds(0, _CH)], osem1).wait()


def kernel(features_list, index_map_list):
    feats = features_list.reshape(_NT * _V, _D)
    offs = (jnp.arange(_NT, dtype=jnp.int32) * _V)[:, None]
    idx = (index_map_list.astype(jnp.int32) + offs).reshape(_TOTAL)
    return _gather_kernel(feats, idx)


# speedup vs baseline: 1.5294x; 1.0003x over previous
import functools

import jax
import jax.numpy as jnp
from jax import lax
from jax.experimental import pallas as pl
from jax.experimental.pallas import tpu as pltpu
from jax.experimental.pallas import tpu_sc as plsc

_NT = 4
_V = 100000
_D = 64
_BT = 150000
_TOTAL = _NT * _BT

_NW = 32
_SUB = 64
_KS = 15
_CH = _SUB * _KS
_NCHUNKS = _TOTAL // _CH
_KMAX = (_NCHUNKS + _NW - 1) // _NW

_mesh = plsc.VectorSubcoreMesh(core_axis_name="c", subcore_axis_name="s")


@functools.partial(
    pl.kernel,
    mesh=_mesh,
    out_type=jax.ShapeDtypeStruct((_TOTAL, _D), jnp.float32),
    scratch_types=[
        pltpu.VMEM((_CH,), jnp.int32),
        pltpu.VMEM((_CH,), jnp.int32),
        pltpu.VMEM((_CH, _D), jnp.float32),
        pltpu.VMEM((_CH, _D), jnp.float32),
        pltpu.SemaphoreType.DMA,
        pltpu.SemaphoreType.DMA,
        pltpu.SemaphoreType.DMA,
    ],
    compiler_params=pltpu.CompilerParams(use_tc_tiling_on_sc=False),
)
def _gather_kernel(feats_hbm, idx_hbm, out_hbm,
                   idx0, idx1, rows0, rows1, gsem, osem0, osem1):
    info = plsc.get_sparse_core_info()
    nc = info.num_cores
    wid = lax.axis_index("s") * nc + lax.axis_index("c")
    count = (_NCHUNKS - wid + _NW - 1) // _NW

    idx_bufs = (idx0, idx1)
    row_bufs = (rows0, rows1)
    osems = (osem0, osem1)

    def do_chunk(k, t):
        idx_v = idx_bufs[t]
        rows_v = row_bufs[t]
        osem = osems[t]
        c = wid + k * _NW
        base = c * _CH

        @pl.when(k < count)
        def _():
            @pl.when(k >= 2)
            def _():
                pltpu.make_async_copy(
                    rows_v, out_hbm.at[pl.ds(0, _CH)], osem
                ).wait()

            pltpu.sync_copy(idx_hbm.at[pl.ds(base, _CH)], idx_v)
            handles = []
            for j in range(_KS):
                handles.append(
                    pltpu.async_copy(
                        feats_hbm.at[idx_v.at[pl.ds(j * _SUB, _SUB)]],
                        rows_v.at[pl.ds(j * _SUB, _SUB)],
                        gsem,
                    )
                )
            for h in handles:
                h.wait()
            pltpu.async_copy(rows_v, out_hbm.at[pl.ds(base, _CH)], osem)

    def pair_body(p, carry):
        do_chunk(2 * p, 0)
        do_chunk(2 * p + 1, 1)
        return carry

    lax.fori_loop(0, (_KMAX + 1) // 2, pair_body, 0)

    pltpu.make_async_copy(rows0, out_hbm.at[pl.ds(0, _CH)], osem0).wait()
    pltpu.make_async_copy(rows1, out_hbm.at[pl.ds(0, _CH)], osem1).wait()


def kernel(features_list, index_map_list):
    feats = features_list.reshape(_NT * _V, _D)
    offs = (jnp.arange(_NT, dtype=jnp.int32) * _V)[:, None]
    idx = (index_map_list.astype(jnp.int32) + offs).reshape(_TOTAL)
    return _gather_kernel(feats, idx)


# speedup vs baseline: 2.2376x; 1.4631x over previous
import functools

import jax
import jax.numpy as jnp
from jax import lax
from jax.experimental import pallas as pl
from jax.experimental.pallas import tpu as pltpu
from jax.experimental.pallas import tpu_sc as plsc

_NT = 4
_V = 100000
_D = 64
_BT = 150000
_TOTAL = _NT * _BT

_NW = 32
_SUB = 64
_KS = 15
_CH = _SUB * _KS
_NCHUNKS = _TOTAL // _CH
_KMAX = (_NCHUNKS + _NW - 1) // _NW

_mesh = plsc.VectorSubcoreMesh(core_axis_name="c", subcore_axis_name="s")


@functools.partial(
    pl.kernel,
    mesh=_mesh,
    out_type=jax.ShapeDtypeStruct((_TOTAL, 2 * _D), jnp.float32),
    scratch_types=[
        pltpu.VMEM((_CH,), jnp.int32),
        pltpu.VMEM((_CH,), jnp.int32),
        pltpu.VMEM((_CH, _D), jnp.float32),
        pltpu.VMEM((_CH, _D), jnp.float32),
        pltpu.SemaphoreType.DMA,
        pltpu.SemaphoreType.DMA,
        pltpu.SemaphoreType.DMA,
    ],
    compiler_params=pltpu.CompilerParams(use_tc_tiling_on_sc=False),
)
def _gather_kernel(feats_hbm, idx_hbm, out_hbm,
                   idx0, idx1, rows0, rows1, gsem, osem0, osem1):
    info = plsc.get_sparse_core_info()
    nc = info.num_cores
    wid = lax.axis_index("s") * nc + lax.axis_index("c")
    count = (_NCHUNKS - wid + _NW - 1) // _NW

    idx_bufs = (idx0, idx1)
    row_bufs = (rows0, rows1)
    osems = (osem0, osem1)

    def do_chunk(k, t):
        idx_v = idx_bufs[t]
        rows_v = row_bufs[t]
        osem = osems[t]
        c = wid + k * _NW
        base = c * _CH

        @pl.when(k < count)
        def _():
            @pl.when(k >= 2)
            def _():
                pltpu.make_async_copy(
                    rows_v, out_hbm.at[pl.ds(0, _CH), pl.ds(0, _D)], osem
                ).wait()

            pltpu.sync_copy(idx_hbm.at[pl.ds(base, _CH)], idx_v)
            handles = []
            for j in range(_KS):
                handles.append(
                    pltpu.async_copy(
                        feats_hbm.at[idx_v.at[pl.ds(j * _SUB, _SUB)]],
                        rows_v.at[pl.ds(j * _SUB, _SUB)],
                        gsem,
                    )
                )
            for h in handles:
                h.wait()
            pltpu.async_copy(
                rows_v, out_hbm.at[pl.ds(base, _CH), pl.ds(0, _D)], osem
            )

    def pair_body(p, carry):
        do_chunk(2 * p, 0)
        do_chunk(2 * p + 1, 1)
        return carry

    lax.fori_loop(0, (_KMAX + 1) // 2, pair_body, 0)

    pltpu.make_async_copy(
        rows0, out_hbm.at[pl.ds(0, _CH), pl.ds(0, _D)], osem0
    ).wait()
    pltpu.make_async_copy(
        rows1, out_hbm.at[pl.ds(0, _CH), pl.ds(0, _D)], osem1
    ).wait()


def kernel(features_list, index_map_list):
    feats = features_list.reshape(_NT * _V, _D)
    offs = (jnp.arange(_NT, dtype=jnp.int32) * _V)[:, None]
    idx = (index_map_list.astype(jnp.int32) + offs).reshape(_TOTAL)
    out_padded = _gather_kernel(feats, idx)
    return out_padded[:, :_D]
